# baseline (speedup 1.0000x reference)
import functools

import jax
import jax.numpy as jnp
from jax import lax
from jax.experimental import pallas as pl
from jax.experimental.pallas import tpu as pltpu
from jax.experimental.pallas import tpu_sc as plsc

BATCH = 16384
HIST = 200
OUT_DIM = 16
TROWS = HIST // 8
NBT = BATCH // 128
NC = 2
NS = 16
NW = NC * NS
BT_W = NBT // NW
STEPS = TROWS * BT_W
OBUF = 8 * 2 * 8 * 128

_mesh = plsc.VectorSubcoreMesh(core_axis_name="c", subcore_axis_name="s")


@functools.partial(
    pl.kernel,
    out_type=jax.ShapeDtypeStruct((HIST * 2 * NBT * 8 * 128,), jnp.float32),
    mesh=_mesh,
    scratch_types=[
        pltpu.VMEM((8, 128), jnp.int32),
        pltpu.VMEM((8, 128), jnp.int32),
        pltpu.VMEM((8, 128, OUT_DIM), jnp.float32),
        pltpu.VMEM((8, 128, OUT_DIM), jnp.float32),
        pltpu.VMEM((OBUF,), jnp.float32),
        pltpu.VMEM((OBUF,), jnp.float32),
    ]
    + [pltpu.SemaphoreType.DMA] * 6,
    compiler_params=pltpu.CompilerParams(
        use_tc_tiling_on_sc=False, needs_layout_passes=False
    ),
)
def _emb_lookup(table_hbm, x4_hbm, out_hbm, i0, i1, r0, r1, o0, o1, *sems):
    idx_v = (i0, i1)
    rows_v = (r0, r1)
    obuf_v = (o0, o1)
    sem_i = sems[0:2]
    sem_g = sems[2:4]
    sem_o = sems[4:6]
    wid = lax.axis_index("s") * NC + lax.axis_index("c")
    bt0 = wid * BT_W

    iota = lax.iota(jnp.int32, 16)
    base_vec = lax.shift_right_logical(iota, 3) * 1024 + lax.bitwise_and(iota, 7) * 128

    def coords(s):
        return lax.div(s, BT_W), bt0 + lax.rem(s, BT_W)

    def idx_dma(s, b):
        trow, btile = coords(s)
        return pltpu.make_async_copy(x4_hbm.at[trow, btile], idx_v[b], sem_i[b])

    def gather_dma(b, t8):
        return pltpu.make_async_copy(
            table_hbm.at[idx_v[b].at[t8]], rows_v[b].at[t8], sem_g[b]
        )

    def out_dma(s, b, t8, eb):
        trow, btile = coords(s)
        off = (((trow * 8 + t8) * 2 + eb) * NBT + btile) * 1024
        return pltpu.make_async_copy(
            obuf_v[b].at[pl.ds(t8 * 2048 + eb * 1024, 1024)],
            out_hbm.at[pl.ds(off, 1024)],
            sem_o[b],
        )

    idx_dma(0, 0).start()
    idx_dma(1, 1).start()
    idx_dma(0, 0).wait()
    for t8 in range(8):
        gather_dma(0, t8).start()

    def phase(i, b):
        bn = 1 - b
        rows = rows_v[b]
        obuf = obuf_v[b]

        for t8 in range(8):
            gather_dma(b, t8).wait()

        @pl.when(i + 2 < STEPS)
        def _():
            idx_dma(i + 2, b).start()

        @pl.when(i + 1 < STEPS)
        def _():
            idx_dma(i + 1, bn).wait()
            for t8 in range(8):
                gather_dma(bn, t8).start()

        @pl.when(i >= 2)
        def _():
            for t8 in range(8):
                for eb in range(2):
                    out_dma(i - 2, b, t8, eb).wait()

        for t8 in range(8):
            base_t8 = base_vec + t8 * 2048

            @plsc.parallel_loop(0, 128, step=1, unroll=32)
            def _(j, t8=t8, base_t8=base_t8):
                plsc.store_scatter(obuf, [base_t8 + j], rows[t8, j, :])

            for eb in range(2):
                out_dma(i, b, t8, eb).start()

    def body(ii, carry):
        phase(2 * ii, 0)
        phase(2 * ii + 1, 1)
        return carry

    lax.fori_loop(0, STEPS // 2, body, 0)

    for s in range(STEPS - 2, STEPS):
        for t8 in range(8):
            for eb in range(2):
                out_dma(s, s % 2, t8, eb).wait()


def kernel(x, W):
    x4 = jnp.transpose(x).reshape(TROWS, 8, NBT, 128).transpose(0, 2, 1, 3)
    out_flat = _emb_lookup(W, x4)
    out5 = out_flat.reshape(HIST, 2, NBT, 8, 128)
    return out5.transpose(2, 4, 0, 1, 3).reshape(BATCH, HIST, OUT_DIM)


# speedup vs baseline: 1.6744x; 1.6744x over previous
import functools

import jax
import jax.numpy as jnp
from jax import lax
from jax.experimental import pallas as pl
from jax.experimental.pallas import tpu as pltpu
from jax.experimental.pallas import tpu_sc as plsc

BATCH = 16384
HIST = 200
OUT_DIM = 16
TROWS = HIST // 8
NBT = BATCH // 128
NC = 2
NS = 16
NW = NC * NS
BT_W = NBT // NW
STEPS = TROWS * BT_W
PAD = 129

_mesh = plsc.VectorSubcoreMesh(core_axis_name="c", subcore_axis_name="s")


@functools.partial(
    pl.kernel,
    out_type=jax.ShapeDtypeStruct((HIST, 2, NBT, 8, 128), jnp.float32),
    mesh=_mesh,
    scratch_types=[
        pltpu.VMEM((8, 128), jnp.int32),
        pltpu.VMEM((8, 128), jnp.int32),
        pltpu.VMEM((8, 128, OUT_DIM), jnp.float32),
        pltpu.VMEM((8, 128, OUT_DIM), jnp.float32),
        pltpu.VMEM((8, OUT_DIM, PAD), jnp.float32),
        pltpu.VMEM((8, OUT_DIM, PAD), jnp.float32),
    ]
    + [pltpu.SemaphoreType.DMA] * 6,
    compiler_params=pltpu.CompilerParams(
        use_tc_tiling_on_sc=False, needs_layout_passes=False
    ),
)
def _emb_lookup(table_hbm, x4_hbm, out_hbm, i0, i1, r0, r1, o0, o1, *sems):
    idx_v = (i0, i1)
    rows_v = (r0, r1)
    obuf_v = (o0, o1)
    sem_i = sems[0:2]
    sem_g = sems[2:4]
    sem_o = sems[4:6]
    wid = lax.axis_index("s") * NC + lax.axis_index("c")
    bt0 = wid * BT_W

    e_vec = lax.iota(jnp.int32, 16)

    def coords(s):
        return lax.div(s, BT_W), bt0 + lax.rem(s, BT_W)

    def idx_dma(s, b):
        trow, btile = coords(s)
        return pltpu.make_async_copy(x4_hbm.at[trow, btile], idx_v[b], sem_i[b])

    def gather_dma(b, t8):
        return pltpu.make_async_copy(
            table_hbm.at[idx_v[b].at[t8]], rows_v[b].at[t8], sem_g[b]
        )

    def out_dma(s, b, t8, eb):
        trow, btile = coords(s)
        return pltpu.make_async_copy(
            obuf_v[b].at[t8, pl.ds(eb * 8, 8), pl.ds(0, 128)],
            out_hbm.at[trow * 8 + t8, eb, btile],
            sem_o[b],
        )

    idx_dma(0, 0).start()
    idx_dma(1, 1).start()
    idx_dma(0, 0).wait()
    for t8 in range(8):
        gather_dma(0, t8).start()

    def phase(i, b):
        bn = 1 - b
        rows = rows_v[b]
        obuf = obuf_v[b]

        for t8 in range(8):
            gather_dma(b, t8).wait()

        @pl.when(i + 2 < STEPS)
        def _():
            idx_dma(i + 2, b).start()

        @pl.when(i + 1 < STEPS)
        def _():
            idx_dma(i + 1, bn).wait()
            for t8 in range(8):
                gather_dma(bn, t8).start()

        @pl.when(i >= 2)
        def _():
            for t8 in range(8):
                for eb in range(2):
                    out_dma(i - 2, b, t8, eb).wait()

        for t8 in range(8):
            ob_t8 = obuf.at[t8]

            @plsc.parallel_loop(0, 128, step=1, unroll=16)
            def _(j, t8=t8, ob_t8=ob_t8):
                plsc.store_scatter(ob_t8, [e_vec, jnp.full((16,), 0, jnp.int32) + j], rows[t8, j, :])

            for eb in range(2):
                out_dma(i, b, t8, eb).start()

    def body(ii, carry):
        phase(2 * ii, 0)
        phase(2 * ii + 1, 1)
        return carry

    lax.fori_loop(0, STEPS // 2, body, 0)

    for s in range(STEPS - 2, STEPS):
        for t8 in range(8):
            for eb in range(2):
                out_dma(s, s % 2, t8, eb).wait()


def kernel(x, W):
    x4 = jnp.transpose(x).reshape(TROWS, 8, NBT, 128).transpose(0, 2, 1, 3)
    out5 = _emb_lookup(W, x4)
    return out5.transpose(2, 4, 0, 1, 3).reshape(BATCH, HIST, OUT_DIM)
